# baseline (speedup 1.0000x reference)
import jax
import jax.numpy as jnp
from jax.experimental import pallas as pl
from jax.experimental.pallas import tpu as pltpu

B, T, F, E, H, HG, C = 4, 2048, 1024, 8, 2048, 64, 1000
TOPK = 2
NP = B * TOPK
TTG = 512
NTG = T // TTG
HT = 512
NH = H // HT
LG = 128

_SQRT2 = 1.4142135623730951


def _gelu(v):
    return 0.5 * v * (1.0 + jax.lax.erf(v / _SQRT2))


def _gating_kernel(x_ref, wg1_ref, bg1_ref, lng_ref, lnb_ref, wg2_ref,
                   bg2_ref, xb_ref, w_out_ref, i_out_ref, acc_ref):
    t = pl.program_id(0)

    @pl.when(t == 0)
    def _():
        acc_ref[...] = jnp.zeros_like(acc_ref)

    xt = x_ref[...]
    xb_ref[...] = xt.astype(jnp.bfloat16)
    acc_ref[0:B, :] += jnp.sum(xt, axis=1)

    @pl.when(t == NTG - 1)
    def _():
        g = acc_ref[0:B, :] / T
        h = jnp.dot(g, wg1_ref[...], preferred_element_type=jnp.float32)
        h = h + bg1_ref[...]
        h = _gelu(h)
        col = jax.lax.broadcasted_iota(jnp.int32, (B, LG), 1)
        real = col < HG
        mu = jnp.sum(h, axis=-1, keepdims=True) / HG
        d = jnp.where(real, h - mu, 0.0)
        var = jnp.sum(d * d, axis=-1, keepdims=True) / HG
        hn = (h - mu) / jnp.sqrt(var + 1e-5) * lng_ref[...] + lnb_ref[...]
        logits = jnp.dot(hn, wg2_ref[...], preferred_element_type=jnp.float32)
        logits = logits + bg2_ref[...]
        logits = jnp.where(col < E, logits, -1e30)
        m = jnp.max(logits, axis=-1, keepdims=True)
        ex = jnp.exp(logits - m)
        rw = ex / jnp.sum(ex, axis=-1, keepdims=True)
        v1 = jnp.max(rw, axis=-1, keepdims=True)
        i1 = jnp.min(jnp.where(rw == v1, col, LG), axis=-1, keepdims=True)
        rw2 = jnp.where(col == i1, -1.0, rw)
        v2 = jnp.max(rw2, axis=-1, keepdims=True)
        i2 = jnp.min(jnp.where(rw2 == v2, col, LG), axis=-1, keepdims=True)
        s = v1 + v2 + 1e-8
        w1 = v1 / s
        w2 = v2 / s
        w_out_ref[...] = jnp.zeros_like(w_out_ref)
        i_out_ref[...] = jnp.zeros_like(i_out_ref)
        w_out_ref[0:B, :] = jnp.where(col == 0, w1,
                                      jnp.where(col == 1, w2, 0.0))
        i_out_ref[0:B, :] = jnp.where(col == 0, i1,
                                      jnp.where(col == 1, i2, 0))


def _expert_kernel(eidx_ref, wts_ref, x_ref, w10_ref, w11_ref, w12_ref,
                   w13_ref, b10_ref, b11_ref, b12_ref, b13_ref, w20_ref,
                   w21_ref, w22_ref, w23_ref, b20_ref, b21_ref, b22_ref,
                   b23_ref, out_ref):
    g = pl.program_id(0)
    ht = pl.program_id(1)
    ones = jnp.full((1, T), 1.0, jnp.bfloat16)
    cdims = (((1,), (1,)), ((), ()))
    w1_refs = (w10_ref, w11_ref, w12_ref, w13_ref)
    b1_refs = (b10_ref, b11_ref, b12_ref, b13_ref)
    w2_refs = (w20_ref, w21_ref, w22_ref, w23_ref)
    b2_refs = (b20_ref, b21_ref, b22_ref, b23_ref)

    parts = []
    for j in range(2 * TOPK):
        xr = x_ref[2 * g + j // TOPK]
        h = jnp.dot(xr, w1_refs[j][0].astype(jnp.bfloat16),
                    preferred_element_type=jnp.float32)
        h = _gelu(h + b1_refs[j][0]).astype(jnp.bfloat16)
        pe = jnp.dot(ones, h, preferred_element_type=jnp.float32) / T
        part = jax.lax.dot_general(pe.astype(jnp.bfloat16),
                                   w2_refs[j][0].astype(jnp.bfloat16),
                                   cdims, preferred_element_type=jnp.float32)
        parts.append(part)

    for r in range(2):
        wa = wts_ref[TOPK * (2 * g + r)]
        wb = wts_ref[TOPK * (2 * g + r) + 1]
        contrib = wa * parts[TOPK * r] + wb * parts[TOPK * r + 1]

        @pl.when(ht == 0)
        def _():
            out_ref[r:r + 1, 0] = (contrib + wa * b2_refs[TOPK * r][0]
                                   + wb * b2_refs[TOPK * r + 1][0])

        @pl.when(ht != 0)
        def _():
            out_ref[r:r + 1, 0] += contrib


def kernel(x, Wg1, bg1, ln_g, ln_b, Wg2, bg2, W1, b1, W2, b2):
    f32 = jnp.float32
    Wg1p = jnp.pad(Wg1, ((0, 0), (0, LG - HG)))
    bg1p = jnp.pad(bg1, (0, LG - HG)).reshape(1, LG)
    lngp = jnp.pad(ln_g, (0, LG - HG)).reshape(1, LG)
    lnbp = jnp.pad(ln_b, (0, LG - HG)).reshape(1, LG)
    Wg2p = jnp.pad(Wg2, ((0, LG - HG), (0, LG - E)))
    bg2p = jnp.pad(bg2, (0, LG - E)).reshape(1, LG)

    xb, w_out, i_out = pl.pallas_call(
        _gating_kernel,
        grid=(NTG,),
        in_specs=[
            pl.BlockSpec((B, TTG, F), lambda t: (0, t, 0)),
            pl.BlockSpec((F, LG), lambda t: (0, 0)),
            pl.BlockSpec((1, LG), lambda t: (0, 0)),
            pl.BlockSpec((1, LG), lambda t: (0, 0)),
            pl.BlockSpec((1, LG), lambda t: (0, 0)),
            pl.BlockSpec((LG, LG), lambda t: (0, 0)),
            pl.BlockSpec((1, LG), lambda t: (0, 0)),
        ],
        out_specs=[
            pl.BlockSpec((B, TTG, F), lambda t: (0, t, 0)),
            pl.BlockSpec((8, LG), lambda t: (0, 0)),
            pl.BlockSpec((8, LG), lambda t: (0, 0)),
        ],
        out_shape=[
            jax.ShapeDtypeStruct((B, T, F), jnp.bfloat16),
            jax.ShapeDtypeStruct((8, LG), f32),
            jax.ShapeDtypeStruct((8, LG), jnp.int32),
        ],
        scratch_shapes=[pltpu.VMEM((8, F), f32)],
    )(x, Wg1p, bg1p, lngp, lnbp, Wg2p, bg2p)

    wflat = w_out[:B, :TOPK].reshape(NP)
    eflat = i_out[:B, :TOPK].reshape(NP)

    b1r = b1.reshape(E, 1, H)
    b2r = b2.reshape(E, 1, C)
    W2t = jnp.swapaxes(W2, 1, 2)

    def w1_spec(j):
        return pl.BlockSpec((1, F, HT), lambda g, ht, eidx, wts:
                            (eidx[2 * TOPK * g + j], 0, ht))

    def b1_spec(j):
        return pl.BlockSpec((1, 1, HT), lambda g, ht, eidx, wts:
                            (eidx[2 * TOPK * g + j], 0, ht))

    def w2_spec(j):
        return pl.BlockSpec((1, C, HT), lambda g, ht, eidx, wts:
                            (eidx[2 * TOPK * g + j], 0, ht))

    def b2_spec(j):
        return pl.BlockSpec((1, 1, C), lambda g, ht, eidx, wts:
                            (eidx[2 * TOPK * g + j], 0, 0))

    grid_spec = pltpu.PrefetchScalarGridSpec(
        num_scalar_prefetch=2,
        grid=(B // 2, NH),
        in_specs=([pl.BlockSpec((B, T, F), lambda g, ht, eidx, wts:
                                (0, 0, 0))]
                  + [w1_spec(j) for j in range(4)]
                  + [b1_spec(j) for j in range(4)]
                  + [w2_spec(j) for j in range(4)]
                  + [b2_spec(j) for j in range(4)]),
        out_specs=pl.BlockSpec((2, 1, C), lambda g, ht, eidx, wts:
                               (g, 0, 0)),
    )

    out = pl.pallas_call(
        _expert_kernel,
        grid_spec=grid_spec,
        out_shape=jax.ShapeDtypeStruct((B, 1, C), f32),
        compiler_params=pltpu.CompilerParams(
            dimension_semantics=("arbitrary", "arbitrary")),
    )(eflat, wflat, xb, W1, W1, W1, W1, b1r, b1r, b1r, b1r,
      W2t, W2t, W2t, W2t, b2r, b2r, b2r, b2r)

    return out.reshape(B, C)


# speedup vs baseline: 1.1307x; 1.1307x over previous
import jax
import jax.numpy as jnp
from jax.experimental import pallas as pl
from jax.experimental.pallas import tpu as pltpu

B, T, F, E, H, HG, C = 4, 2048, 1024, 8, 2048, 64, 1000
TOPK = 2
NP = B * TOPK
TTG = 512
NTG = T // TTG
HT = 512
NH = H // HT

_SQRT2 = 1.4142135623730951


def _gelu(v):
    return 0.5 * v * (1.0 + jax.lax.erf(v / _SQRT2))


def _gating_kernel(x_ref, wg1_ref, bg1_ref, lng_ref, lnb_ref, wg2_ref,
                   bg2_ref, xb_ref, ei_ref, wt_ref, acc_ref):
    t = pl.program_id(0)

    @pl.when(t == 0)
    def _():
        acc_ref[...] = jnp.zeros_like(acc_ref)

    xt = x_ref[...]
    xb_ref[...] = xt.astype(jnp.bfloat16)
    acc_ref[0:B, :] += jnp.sum(xt, axis=1)

    @pl.when(t == NTG - 1)
    def _():
        cd = (((1,), (1,)), ((), ()))
        g = acc_ref[0:B, :] / T
        h = jax.lax.dot_general(g, wg1_ref[...], cd,
                                preferred_element_type=jnp.float32)
        h = h + bg1_ref[...]
        h = _gelu(h)
        mu = jnp.mean(h, axis=-1, keepdims=True)
        d = h - mu
        var = jnp.mean(d * d, axis=-1, keepdims=True)
        hn = d / jnp.sqrt(var + 1e-5) * lng_ref[...] + lnb_ref[...]
        logits = jax.lax.dot_general(hn, wg2_ref[...], cd,
                                     preferred_element_type=jnp.float32)
        logits = logits + bg2_ref[...]
        m = jnp.max(logits, axis=-1, keepdims=True)
        ex = jnp.exp(logits - m)
        rw = ex / jnp.sum(ex, axis=-1, keepdims=True)
        col = jax.lax.broadcasted_iota(jnp.int32, (B, E), 1)
        v1 = jnp.max(rw, axis=-1, keepdims=True)
        i1 = jnp.min(jnp.where(rw == v1, col, E), axis=-1, keepdims=True)
        rw2 = jnp.where(col == i1, -1.0, rw)
        v2 = jnp.max(rw2, axis=-1, keepdims=True)
        i2 = jnp.min(jnp.where(rw2 == v2, col, E), axis=-1, keepdims=True)
        s = v1 + v2 + 1e-8
        w1 = v1 / s
        w2 = v2 / s
        rowv = jax.lax.broadcasted_iota(jnp.int32, (B, 1), 0)
        for p in range(NP):
            b, k = divmod(p, TOPK)
            iv = i1 if k == 0 else i2
            wv = w1 if k == 0 else w2
            onb = rowv == b
            ei_ref[p] = jnp.sum(jnp.where(onb, iv, 0))
            wt_ref[p] = jnp.sum(jnp.where(onb, wv, 0.0))


def _expert_kernel(eidx_ref, wts_ref, x_ref, w1a_ref, w1b_ref, w2a_ref,
                   w2b_ref, b1_ref, b2_ref, out_ref):
    b = pl.program_id(0)
    ht = pl.program_id(1)
    xr = x_ref[b]
    ea = eidx_ref[TOPK * b]
    eb = eidx_ref[TOPK * b + 1]
    wa = wts_ref[TOPK * b]
    wb = wts_ref[TOPK * b + 1]
    ones = jnp.full((1, T), 1.0, jnp.bfloat16)
    cdims = (((1,), (1,)), ((), ()))
    hsl = pl.ds(ht * HT, HT)

    ha = jnp.dot(xr, w1a_ref[0].astype(jnp.bfloat16),
                 preferred_element_type=jnp.float32)
    hb = jnp.dot(xr, w1b_ref[0].astype(jnp.bfloat16),
                 preferred_element_type=jnp.float32)
    ha = _gelu(ha + b1_ref[pl.ds(ea, 1), hsl])
    hb = _gelu(hb + b1_ref[pl.ds(eb, 1), hsl])
    pea = jnp.dot(ones, ha.astype(jnp.bfloat16),
                  preferred_element_type=jnp.float32) / T
    peb = jnp.dot(ones, hb.astype(jnp.bfloat16),
                  preferred_element_type=jnp.float32) / T
    parta = jax.lax.dot_general(pea.astype(jnp.bfloat16),
                                w2a_ref[0].astype(jnp.bfloat16),
                                cdims, preferred_element_type=jnp.float32)
    partb = jax.lax.dot_general(peb.astype(jnp.bfloat16),
                                w2b_ref[0].astype(jnp.bfloat16),
                                cdims, preferred_element_type=jnp.float32)
    contrib = wa * parta + wb * partb

    @pl.when(ht == 0)
    def _():
        out_ref[0] = (contrib + wa * b2_ref[pl.ds(ea, 1), :]
                      + wb * b2_ref[pl.ds(eb, 1), :])

    @pl.when(ht != 0)
    def _():
        out_ref[0] += contrib


def kernel(x, Wg1, bg1, ln_g, ln_b, Wg2, bg2, W1, b1, W2, b2):
    f32 = jnp.float32
    Wg1t = jnp.swapaxes(Wg1, 0, 1)
    Wg2t = jnp.swapaxes(Wg2, 0, 1)
    W2t = jnp.swapaxes(W2, 1, 2)
    bg1r = bg1.reshape(1, HG)
    lngr = ln_g.reshape(1, HG)
    lnbr = ln_b.reshape(1, HG)
    bg2r = bg2.reshape(1, E)

    xb, eidx, wts = pl.pallas_call(
        _gating_kernel,
        grid=(NTG,),
        in_specs=[
            pl.BlockSpec((B, TTG, F), lambda t: (0, t, 0)),
            pl.BlockSpec((HG, F), lambda t: (0, 0)),
            pl.BlockSpec((1, HG), lambda t: (0, 0)),
            pl.BlockSpec((1, HG), lambda t: (0, 0)),
            pl.BlockSpec((1, HG), lambda t: (0, 0)),
            pl.BlockSpec((E, HG), lambda t: (0, 0)),
            pl.BlockSpec((1, E), lambda t: (0, 0)),
        ],
        out_specs=[
            pl.BlockSpec((B, TTG, F), lambda t: (0, t, 0)),
            pl.BlockSpec(memory_space=pltpu.SMEM),
            pl.BlockSpec(memory_space=pltpu.SMEM),
        ],
        out_shape=[
            jax.ShapeDtypeStruct((B, T, F), jnp.bfloat16),
            jax.ShapeDtypeStruct((NP,), jnp.int32),
            jax.ShapeDtypeStruct((NP,), f32),
        ],
        scratch_shapes=[pltpu.VMEM((8, F), f32)],
    )(x, Wg1t, bg1r, lngr, lnbr, Wg2t, bg2r)

    grid_spec = pltpu.PrefetchScalarGridSpec(
        num_scalar_prefetch=2,
        grid=(B, NH),
        in_specs=[
            pl.BlockSpec((B, T, F), lambda b, ht, eidx, wts: (0, 0, 0)),
            pl.BlockSpec((1, F, HT), lambda b, ht, eidx, wts:
                         (eidx[TOPK * b], 0, ht)),
            pl.BlockSpec((1, F, HT), lambda b, ht, eidx, wts:
                         (eidx[TOPK * b + 1], 0, ht)),
            pl.BlockSpec((1, C, HT), lambda b, ht, eidx, wts:
                         (eidx[TOPK * b], 0, ht)),
            pl.BlockSpec((1, C, HT), lambda b, ht, eidx, wts:
                         (eidx[TOPK * b + 1], 0, ht)),
            pl.BlockSpec((E, H), lambda b, ht, eidx, wts: (0, 0)),
            pl.BlockSpec((E, C), lambda b, ht, eidx, wts: (0, 0)),
        ],
        out_specs=pl.BlockSpec((1, 1, C), lambda b, ht, eidx, wts:
                               (b, 0, 0)),
    )

    out = pl.pallas_call(
        _expert_kernel,
        grid_spec=grid_spec,
        out_shape=jax.ShapeDtypeStruct((B, 1, C), f32),
        compiler_params=pltpu.CompilerParams(
            dimension_semantics=("arbitrary", "arbitrary")),
    )(eidx, wts, xb, W1, W1, W2t, W2t, b1, b2)

    return out.reshape(B, C)


# speedup vs baseline: 1.1656x; 1.0308x over previous
import jax
import jax.numpy as jnp
from jax.experimental import pallas as pl
from jax.experimental.pallas import tpu as pltpu

B, T, F, E, H, HG, C = 4, 2048, 1024, 8, 2048, 64, 1000
TOPK = 2
NP = B * TOPK
TTG = 512
NTG = T // TTG
HT = 1024
NH = H // HT

_SQRT2 = 1.4142135623730951


def _gelu(v):
    return 0.5 * v * (1.0 + jax.lax.erf(v / _SQRT2))


def _gating_kernel(x_ref, wg1_ref, bg1_ref, lng_ref, lnb_ref, wg2_ref,
                   bg2_ref, xb_ref, ei_ref, wt_ref, acc_ref):
    t = pl.program_id(0)

    @pl.when(t == 0)
    def _():
        acc_ref[...] = jnp.zeros_like(acc_ref)

    xt = x_ref[...]
    xb_ref[...] = xt.astype(jnp.bfloat16)
    acc_ref[0:B, :] += jnp.sum(xt, axis=1)

    @pl.when(t == NTG - 1)
    def _():
        cd = (((1,), (1,)), ((), ()))
        g = acc_ref[0:B, :] / T
        h = jax.lax.dot_general(g, wg1_ref[...], cd,
                                preferred_element_type=jnp.float32)
        h = h + bg1_ref[...]
        h = _gelu(h)
        mu = jnp.mean(h, axis=-1, keepdims=True)
        d = h - mu
        var = jnp.mean(d * d, axis=-1, keepdims=True)
        hn = d / jnp.sqrt(var + 1e-5) * lng_ref[...] + lnb_ref[...]
        logits = jax.lax.dot_general(hn, wg2_ref[...], cd,
                                     preferred_element_type=jnp.float32)
        logits = logits + bg2_ref[...]
        m = jnp.max(logits, axis=-1, keepdims=True)
        ex = jnp.exp(logits - m)
        rw = ex / jnp.sum(ex, axis=-1, keepdims=True)
        col = jax.lax.broadcasted_iota(jnp.int32, (B, E), 1)
        v1 = jnp.max(rw, axis=-1, keepdims=True)
        i1 = jnp.min(jnp.where(rw == v1, col, E), axis=-1, keepdims=True)
        rw2 = jnp.where(col == i1, -1.0, rw)
        v2 = jnp.max(rw2, axis=-1, keepdims=True)
        i2 = jnp.min(jnp.where(rw2 == v2, col, E), axis=-1, keepdims=True)
        s = v1 + v2 + 1e-8
        w1 = v1 / s
        w2 = v2 / s
        rowv = jax.lax.broadcasted_iota(jnp.int32, (B, 1), 0)
        for p in range(NP):
            b, k = divmod(p, TOPK)
            iv = i1 if k == 0 else i2
            wv = w1 if k == 0 else w2
            onb = rowv == b
            ei_ref[p] = jnp.sum(jnp.where(onb, iv, 0))
            wt_ref[p] = jnp.sum(jnp.where(onb, wv, 0.0))


def _expert_kernel(eidx_ref, wts_ref, x_ref, w1a_ref, w1b_ref, w2a_ref,
                   w2b_ref, b1_ref, b2_ref, out_ref):
    b = pl.program_id(0)
    ht = pl.program_id(1)
    xr = x_ref[0]
    ea = eidx_ref[TOPK * b]
    eb = eidx_ref[TOPK * b + 1]
    wa = wts_ref[TOPK * b]
    wb = wts_ref[TOPK * b + 1]
    ones = jnp.full((1, T), 1.0, jnp.bfloat16)
    cdims = (((1,), (1,)), ((), ()))
    hsl = pl.ds(ht * HT, HT)

    ha = jnp.dot(xr, w1a_ref[0].astype(jnp.bfloat16),
                 preferred_element_type=jnp.float32)
    hb = jnp.dot(xr, w1b_ref[0].astype(jnp.bfloat16),
                 preferred_element_type=jnp.float32)
    ha = _gelu(ha + b1_ref[pl.ds(ea, 1), hsl])
    hb = _gelu(hb + b1_ref[pl.ds(eb, 1), hsl])
    pea = jnp.dot(ones, ha.astype(jnp.bfloat16),
                  preferred_element_type=jnp.float32) / T
    peb = jnp.dot(ones, hb.astype(jnp.bfloat16),
                  preferred_element_type=jnp.float32) / T
    parta = jax.lax.dot_general(pea.astype(jnp.bfloat16),
                                w2a_ref[0].astype(jnp.bfloat16),
                                cdims, preferred_element_type=jnp.float32)
    partb = jax.lax.dot_general(peb.astype(jnp.bfloat16),
                                w2b_ref[0].astype(jnp.bfloat16),
                                cdims, preferred_element_type=jnp.float32)
    contrib = wa * parta + wb * partb

    @pl.when(ht == 0)
    def _():
        out_ref[0] = (contrib + wa * b2_ref[pl.ds(ea, 1), :]
                      + wb * b2_ref[pl.ds(eb, 1), :])

    @pl.when(ht != 0)
    def _():
        out_ref[0] += contrib


def kernel(x, Wg1, bg1, ln_g, ln_b, Wg2, bg2, W1, b1, W2, b2):
    f32 = jnp.float32
    Wg1t = jnp.swapaxes(Wg1, 0, 1)
    Wg2t = jnp.swapaxes(Wg2, 0, 1)
    W2t = jnp.swapaxes(W2, 1, 2)
    bg1r = bg1.reshape(1, HG)
    lngr = ln_g.reshape(1, HG)
    lnbr = ln_b.reshape(1, HG)
    bg2r = bg2.reshape(1, E)

    xb, eidx, wts = pl.pallas_call(
        _gating_kernel,
        grid=(NTG,),
        in_specs=[
            pl.BlockSpec((B, TTG, F), lambda t: (0, t, 0)),
            pl.BlockSpec((HG, F), lambda t: (0, 0)),
            pl.BlockSpec((1, HG), lambda t: (0, 0)),
            pl.BlockSpec((1, HG), lambda t: (0, 0)),
            pl.BlockSpec((1, HG), lambda t: (0, 0)),
            pl.BlockSpec((E, HG), lambda t: (0, 0)),
            pl.BlockSpec((1, E), lambda t: (0, 0)),
        ],
        out_specs=[
            pl.BlockSpec((B, TTG, F), lambda t: (0, t, 0)),
            pl.BlockSpec(memory_space=pltpu.SMEM),
            pl.BlockSpec(memory_space=pltpu.SMEM),
        ],
        out_shape=[
            jax.ShapeDtypeStruct((B, T, F), jnp.bfloat16),
            jax.ShapeDtypeStruct((NP,), jnp.int32),
            jax.ShapeDtypeStruct((NP,), f32),
        ],
        scratch_shapes=[pltpu.VMEM((8, F), f32)],
    )(x, Wg1t, bg1r, lngr, lnbr, Wg2t, bg2r)

    grid_spec = pltpu.PrefetchScalarGridSpec(
        num_scalar_prefetch=2,
        grid=(B, NH),
        in_specs=[
            pl.BlockSpec((1, T, F), lambda b, ht, eidx, wts: (b, 0, 0)),
            pl.BlockSpec((1, F, HT), lambda b, ht, eidx, wts:
                         (eidx[TOPK * b], 0, ht)),
            pl.BlockSpec((1, F, HT), lambda b, ht, eidx, wts:
                         (eidx[TOPK * b + 1], 0, ht)),
            pl.BlockSpec((1, C, HT), lambda b, ht, eidx, wts:
                         (eidx[TOPK * b], 0, ht)),
            pl.BlockSpec((1, C, HT), lambda b, ht, eidx, wts:
                         (eidx[TOPK * b + 1], 0, ht)),
            pl.BlockSpec((E, H), lambda b, ht, eidx, wts: (0, 0)),
            pl.BlockSpec((E, C), lambda b, ht, eidx, wts: (0, 0)),
        ],
        out_specs=pl.BlockSpec((1, 1, C), lambda b, ht, eidx, wts:
                               (b, 0, 0)),
    )

    out = pl.pallas_call(
        _expert_kernel,
        grid_spec=grid_spec,
        out_shape=jax.ShapeDtypeStruct((B, 1, C), f32),
        compiler_params=pltpu.CompilerParams(
            dimension_semantics=("arbitrary", "arbitrary")),
    )(eidx, wts, xb, W1, W1, W2t, W2t, b1, b2)

    return out.reshape(B, C)
